# baseline (speedup 1.0000x reference)
import jax
import jax.numpy as jnp
from jax import lax
from jax.experimental import pallas as pl
from jax.experimental.pallas import tpu as pltpu

VOCAB = 100000
EMBED = 64
HIDDEN = 128
CTX = 200

BLK = 4000
KBLKS = VOCAB // BLK

NEMB = 16
SLOTS = 208
GSTEPS = SLOTS // NEMB
NPAD = SLOTS - CTX


def _fused(idx_pad, emb, W1, b1, W2, b2_blocked):
  def body(idx_ref, *refs):
    e_refs = refs[:NEMB]
    ez_ref = refs[NEMB]
    w1_ref, b1_ref, w2_ref, b2_ref, out_ref, acc_scr, h_scr, ms_scr = \
        refs[NEMB + 1:]
    i = pl.program_id(0)

    @pl.when(i < GSTEPS)
    def _():
      base = NEMB * jnp.minimum(i, GSTEPS - 1)
      s = jnp.zeros((1, EMBED), jnp.float32)
      for j in range(NEMB):
        r = idx_ref[base + j] % 8
        s = s + e_refs[j][pl.ds(r, 1), :]
      acc_scr[...] = jnp.where(i == 0, s, acc_scr[...] + s)

    @pl.when(i == GSTEPS - 1)
    def _():
      ctx = acc_scr[...] - jnp.float32(NPAD) * ez_ref[pl.ds(0, 1), :]
      h = lax.dot_general(
          ctx, w1_ref[...], (((1,), (1,)), ((), ())),
          preferred_element_type=jnp.float32,
      ) + b1_ref[...]
      h_scr[...] = jnp.maximum(h, 0.0)
      ms_scr[0] = -jnp.inf
      ms_scr[1] = 0.0

    @pl.when((i >= GSTEPS) & (i < GSTEPS + KBLKS))
    def _():
      h = h_scr[...]
      logits = lax.dot_general(
          h, w2_ref[...], (((1,), (1,)), ((), ())),
          preferred_element_type=jnp.float32,
      ) + b2_ref[0]
      m = ms_scr[0]
      s = ms_scr[1]
      bm = jnp.max(logits)
      new_m = jnp.maximum(m, bm)
      ms_scr[0] = new_m
      ms_scr[1] = s * jnp.exp(m - new_m) + jnp.sum(jnp.exp(logits - new_m))
      out_ref[pl.ds(i - GSTEPS, 1), :] = logits

    @pl.when(i == GSTEPS + KBLKS)
    def _():
      lse = ms_scr[0] + jnp.log(ms_scr[1])
      out_ref[...] = out_ref[...] - lse

  def emb_spec(j):
    return pl.BlockSpec(
        (8, EMBED),
        lambda i, idx_ref, j=j: (
            idx_ref[NEMB * jnp.minimum(i, GSTEPS - 1) + j] // 8, 0),
    )

  wclamp = lambda i, idx_ref: (jnp.clip(i - GSTEPS, 0, KBLKS - 1), 0)

  grid_spec = pltpu.PrefetchScalarGridSpec(
      num_scalar_prefetch=1,
      grid=(GSTEPS + KBLKS + 1,),
      in_specs=[
          *[emb_spec(j) for j in range(NEMB)],
          pl.BlockSpec((8, EMBED), lambda i, idx_ref: (0, 0)),
          pl.BlockSpec((HIDDEN, EMBED), lambda i, idx_ref: (0, 0)),
          pl.BlockSpec((1, HIDDEN), lambda i, idx_ref: (0, 0)),
          pl.BlockSpec((BLK, HIDDEN), wclamp),
          pl.BlockSpec(
              (1, 1, BLK),
              lambda i, idx_ref: (jnp.clip(i - GSTEPS, 0, KBLKS - 1), 0, 0)),
      ],
      out_specs=pl.BlockSpec((KBLKS, BLK), lambda i, idx_ref: (0, 0)),
      scratch_shapes=[
          pltpu.VMEM((1, EMBED), jnp.float32),
          pltpu.VMEM((1, HIDDEN), jnp.float32),
          pltpu.SMEM((2,), jnp.float32),
      ],
  )

  return pl.pallas_call(
      body,
      grid_spec=grid_spec,
      out_shape=jax.ShapeDtypeStruct((KBLKS, BLK), jnp.float32),
  )(idx_pad, *([emb] * NEMB), emb, W1, b1, W2, b2_blocked)


def kernel(inputs, emb, W1, b1, W2, b2):
  idx = inputs.astype(jnp.int32)
  idx_pad = jnp.concatenate([idx, jnp.zeros((SLOTS - CTX,), jnp.int32)])
  b1r = b1.astype(jnp.float32).reshape(1, HIDDEN)
  b2r = b2.astype(jnp.float32).reshape(KBLKS, 1, BLK)
  out = _fused(idx_pad, emb, W1, b1r, W2, b2r)
  return out.reshape(1, VOCAB)


# speedup vs baseline: 1.0415x; 1.0415x over previous
import jax
import jax.numpy as jnp
from jax import lax
from jax.experimental import pallas as pl
from jax.experimental.pallas import tpu as pltpu

VOCAB = 100000
EMBED = 64
HIDDEN = 128
CTX = 200

BLK = 4000
KBLKS = VOCAB // BLK

GOPS = 52
GSTEPS = 4
SLOTS = GOPS * GSTEPS
NPAD = SLOTS - CTX


def _gather_h(idx_pad, emb, W1, b1):
  def body(idx_ref, *refs):
    e_refs = refs[:GOPS]
    ez_ref = refs[GOPS]
    w1_ref, b1_ref, out_ref, acc_scr = refs[GOPS + 1:]
    i = pl.program_id(0)

    base = GOPS * i
    s = jnp.zeros((1, EMBED), jnp.float32)
    for j in range(GOPS):
      r = idx_ref[base + j] % 8
      s = s + e_refs[j][pl.ds(r, 1), :]
    acc_scr[...] = jnp.where(i == 0, s, acc_scr[...] + s)

    @pl.when(i == GSTEPS - 1)
    def _():
      ctx = acc_scr[...] - jnp.float32(NPAD) * ez_ref[pl.ds(0, 1), :]
      h = lax.dot_general(
          ctx, w1_ref[...], (((1,), (1,)), ((), ())),
          preferred_element_type=jnp.float32,
      ) + b1_ref[...]
      out_ref[...] = jnp.maximum(h, 0.0)

  def emb_spec(j):
    return pl.BlockSpec(
        (8, EMBED),
        lambda i, idx_ref, j=j: (idx_ref[GOPS * i + j] // 8, 0),
    )

  grid_spec = pltpu.PrefetchScalarGridSpec(
      num_scalar_prefetch=1,
      grid=(GSTEPS,),
      in_specs=[
          *[emb_spec(j) for j in range(GOPS)],
          pl.BlockSpec((8, EMBED), lambda i, idx_ref: (0, 0)),
          pl.BlockSpec((HIDDEN, EMBED), lambda i, idx_ref: (0, 0)),
          pl.BlockSpec((1, HIDDEN), lambda i, idx_ref: (0, 0)),
      ],
      out_specs=pl.BlockSpec((1, HIDDEN), lambda i, idx_ref: (0, 0)),
      scratch_shapes=[
          pltpu.VMEM((1, EMBED), jnp.float32),
      ],
  )

  return pl.pallas_call(
      body,
      grid_spec=grid_spec,
      out_shape=jax.ShapeDtypeStruct((1, HIDDEN), jnp.float32),
  )(idx_pad, *([emb] * GOPS), emb, W1, b1)


def _mlp(h, W2, b2_blocked):
  def body(h_ref, w2_ref, b2_ref, out_ref, ms_scr):
    i = pl.program_id(0)

    @pl.when(i == 0)
    def _():
      ms_scr[0] = -jnp.inf
      ms_scr[1] = 0.0

    @pl.when(i < KBLKS)
    def _():
      logits = lax.dot_general(
          h_ref[...], w2_ref[...], (((1,), (1,)), ((), ())),
          preferred_element_type=jnp.float32,
      ) + b2_ref[0]
      m = ms_scr[0]
      s = ms_scr[1]
      bm = jnp.max(logits)
      new_m = jnp.maximum(m, bm)
      ms_scr[0] = new_m
      ms_scr[1] = s * jnp.exp(m - new_m) + jnp.sum(jnp.exp(logits - new_m))
      out_ref[pl.ds(i, 1), :] = logits

    @pl.when(i == KBLKS)
    def _():
      lse = ms_scr[0] + jnp.log(ms_scr[1])
      out_ref[...] = out_ref[...] - lse

  return pl.pallas_call(
      body,
      grid=(KBLKS + 1,),
      in_specs=[
          pl.BlockSpec((1, HIDDEN), lambda i: (0, 0)),
          pl.BlockSpec((BLK, HIDDEN), lambda i: (jnp.minimum(i, KBLKS - 1), 0)),
          pl.BlockSpec((1, 1, BLK), lambda i: (jnp.minimum(i, KBLKS - 1), 0, 0)),
      ],
      out_specs=pl.BlockSpec((KBLKS, BLK), lambda i: (0, 0)),
      out_shape=jax.ShapeDtypeStruct((KBLKS, BLK), jnp.float32),
      scratch_shapes=[
          pltpu.SMEM((2,), jnp.float32),
      ],
  )(h, W2, b2_blocked)


def kernel(inputs, emb, W1, b1, W2, b2):
  idx = inputs.astype(jnp.int32)
  idx_pad = jnp.concatenate([idx, jnp.zeros((SLOTS - CTX,), jnp.int32)])
  b1r = b1.astype(jnp.float32).reshape(1, HIDDEN)
  b2r = b2.astype(jnp.float32).reshape(KBLKS, 1, BLK)
  h = _gather_h(idx_pad, emb, W1, b1r)
  out = _mlp(h, W2, b2r)
  return out.reshape(1, VOCAB)


# speedup vs baseline: 1.1953x; 1.1478x over previous
import jax
import jax.numpy as jnp
from jax import lax
from jax.experimental import pallas as pl
from jax.experimental.pallas import tpu as pltpu

VOCAB = 100000
EMBED = 64
HIDDEN = 128
CTX = 200

BLK = 5000
KBLKS = VOCAB // BLK

NQ = 8


def _fused(idx, emb, W1, b1, W2, b2_blocked):
  def body(idx_ref, emb_ref, w1_ref, b1_ref, w2_ref, b2_ref, out_ref,
           rows_scr, h_scr, ms_scr, sem):
    i = pl.program_id(0)

    @pl.when(i == 0)
    def _():
      copies = []
      for r in range(CTX):
        v = idx_ref[r]
        cp = pltpu.make_async_copy(
            emb_ref.at[pl.ds(v, 1)], rows_scr.at[pl.ds(r, 1)],
            sem.at[r % NQ]
        )
        cp.start()
        copies.append(cp)
      for cp in copies:
        cp.wait()
      ctx = jnp.sum(rows_scr[...], axis=0, keepdims=True)
      h = lax.dot_general(
          ctx, w1_ref[...], (((1,), (1,)), ((), ())),
          preferred_element_type=jnp.float32,
      ) + b1_ref[...]
      h_scr[...] = jnp.maximum(h, 0.0)
      ms_scr[0] = -jnp.inf
      ms_scr[1] = 0.0

    @pl.when(i < KBLKS)
    def _():
      h = h_scr[...]
      logits = lax.dot_general(
          h, w2_ref[...], (((1,), (1,)), ((), ())),
          preferred_element_type=jnp.float32,
      ) + b2_ref[0]
      m = ms_scr[0]
      s = ms_scr[1]
      bm = jnp.max(logits)
      new_m = jnp.maximum(m, bm)
      ms_scr[0] = new_m
      ms_scr[1] = s * jnp.exp(m - new_m) + jnp.sum(jnp.exp(logits - new_m))
      out_ref[pl.ds(i, 1), :] = logits

    @pl.when(i == KBLKS)
    def _():
      lse = ms_scr[0] + jnp.log(ms_scr[1])
      out_ref[...] = out_ref[...] - lse

  return pl.pallas_call(
      body,
      grid=(KBLKS + 1,),
      in_specs=[
          pl.BlockSpec(memory_space=pltpu.SMEM),
          pl.BlockSpec(memory_space=pltpu.MemorySpace.HBM),
          pl.BlockSpec((HIDDEN, EMBED), lambda i: (0, 0)),
          pl.BlockSpec((1, HIDDEN), lambda i: (0, 0)),
          pl.BlockSpec((BLK, HIDDEN), lambda i: (jnp.minimum(i, KBLKS - 1), 0)),
          pl.BlockSpec((1, 1, BLK), lambda i: (jnp.minimum(i, KBLKS - 1), 0, 0)),
      ],
      out_specs=pl.BlockSpec((KBLKS, BLK), lambda i: (0, 0)),
      out_shape=jax.ShapeDtypeStruct((KBLKS, BLK), jnp.float32),
      scratch_shapes=[
          pltpu.VMEM((CTX, EMBED), jnp.float32),
          pltpu.VMEM((1, HIDDEN), jnp.float32),
          pltpu.SMEM((2,), jnp.float32),
          pltpu.SemaphoreType.DMA((NQ,)),
      ],
  )(idx, emb, W1, b1, W2, b2_blocked)


def kernel(inputs, emb, W1, b1, W2, b2):
  idx = inputs.astype(jnp.int32)
  b1r = b1.astype(jnp.float32).reshape(1, HIDDEN)
  b2r = b2.astype(jnp.float32).reshape(KBLKS, 1, BLK)
  out = _fused(idx, emb, W1, b1r, W2, b2r)
  return out.reshape(1, VOCAB)


# speedup vs baseline: 1.3215x; 1.1056x over previous
import jax
import jax.numpy as jnp
from jax import lax
from jax.experimental import pallas as pl
from jax.experimental.pallas import tpu as pltpu

VOCAB = 100000
EMBED = 64
HIDDEN = 128
CTX = 200

BLK = 20000
KBLKS = VOCAB // BLK

NQ = 8


def _fused(idx, emb, W1, b1, W2, b2_blocked):
  def body(idx_ref, emb_ref, w1_ref, b1_ref, w2_ref, b2_ref, out_ref,
           rows_scr, h_scr, ms_scr, sem):
    i = pl.program_id(0)

    @pl.when(i == 0)
    def _():
      copies = []
      for r in range(CTX):
        v = idx_ref[r]
        cp = pltpu.make_async_copy(
            emb_ref.at[pl.ds(v, 1)], rows_scr.at[pl.ds(r, 1)],
            sem.at[r % NQ]
        )
        cp.start()
        copies.append(cp)
      for cp in copies:
        cp.wait()
      ctx = jnp.sum(rows_scr[...], axis=0, keepdims=True)
      h = lax.dot_general(
          ctx, w1_ref[...], (((1,), (1,)), ((), ())),
          preferred_element_type=jnp.float32,
      ) + b1_ref[...]
      h_scr[...] = jnp.maximum(h, 0.0)
      ms_scr[0] = -jnp.inf
      ms_scr[1] = 0.0

    @pl.when(i < KBLKS)
    def _():
      h = h_scr[...]
      logits = lax.dot_general(
          h, w2_ref[...], (((1,), (1,)), ((), ())),
          preferred_element_type=jnp.float32,
      ) + b2_ref[0]
      m = ms_scr[0]
      s = ms_scr[1]
      bm = jnp.max(logits)
      new_m = jnp.maximum(m, bm)
      ms_scr[0] = new_m
      ms_scr[1] = s * jnp.exp(m - new_m) + jnp.sum(jnp.exp(logits - new_m))
      out_ref[pl.ds(i, 1), :] = logits

    @pl.when(i == KBLKS)
    def _():
      lse = ms_scr[0] + jnp.log(ms_scr[1])
      out_ref[...] = out_ref[...] - lse

  return pl.pallas_call(
      body,
      grid=(KBLKS + 1,),
      in_specs=[
          pl.BlockSpec(memory_space=pltpu.SMEM),
          pl.BlockSpec(memory_space=pltpu.MemorySpace.HBM),
          pl.BlockSpec((HIDDEN, EMBED), lambda i: (0, 0)),
          pl.BlockSpec((1, HIDDEN), lambda i: (0, 0)),
          pl.BlockSpec((BLK, HIDDEN), lambda i: (jnp.minimum(i, KBLKS - 1), 0)),
          pl.BlockSpec((1, 1, BLK), lambda i: (jnp.minimum(i, KBLKS - 1), 0, 0)),
      ],
      out_specs=pl.BlockSpec((KBLKS, BLK), lambda i: (0, 0)),
      out_shape=jax.ShapeDtypeStruct((KBLKS, BLK), jnp.float32),
      scratch_shapes=[
          pltpu.VMEM((CTX, EMBED), jnp.float32),
          pltpu.VMEM((1, HIDDEN), jnp.float32),
          pltpu.SMEM((2,), jnp.float32),
          pltpu.SemaphoreType.DMA((NQ,)),
      ],
  )(idx, emb, W1, b1, W2, b2_blocked)


def kernel(inputs, emb, W1, b1, W2, b2):
  idx = inputs.astype(jnp.int32)
  b1r = b1.astype(jnp.float32).reshape(1, HIDDEN)
  b2r = b2.astype(jnp.float32).reshape(KBLKS, 1, BLK)
  out = _fused(idx, emb, W1, b1r, W2, b2r)
  return out.reshape(1, VOCAB)
